# baseline (speedup 1.0000x reference)
import jax
import jax.numpy as jnp
from jax.experimental import pallas as pl
from jax.experimental.pallas import tpu as pltpu

_REG_MAX = 16
_NC = 80
_NREG = 4 * (_REG_MAX + 1)
_STRIDES = (32, 16, 8)
_HWS = ((20, 20), (40, 40), (80, 80))
_OFFSET = 0.5


def _scale_compute(H, W, i, gates_col, feat_ref,
                   fcwc, fcbc, ccw, rcw, scat, bcat,
                   cpw, cpb, rpw, rpb):
    C = feat_ref.shape[1]
    L = H * W
    bf = jnp.bfloat16
    cx = jax.lax.broadcasted_iota(jnp.int32, (1, L), 1) % W
    proj = jax.lax.broadcasted_iota(
        jnp.int32, (_REG_MAX + 1, 1), 0).astype(jnp.float32)

    def conv3x3(xb, w_ref):
        zrow = jnp.zeros((C, W), bf)
        x3 = jnp.concatenate([
            jnp.concatenate([zrow, xb[:, :L - W]], axis=1),
            xb,
            jnp.concatenate([xb[:, W:], zrow], axis=1),
        ], axis=0)
        P = [jnp.dot(w_ref[kx], x3, preferred_element_type=jnp.float32)
             for kx in range(3)]
        left = pltpu.roll(P[0], 1, axis=1)
        right = pltpu.roll(P[2], L - 1, axis=1)
        return (P[1] + jnp.where(cx >= 1, left, 0.0)
                + jnp.where(cx <= W - 2, right, 0.0))

    feat = feat_ref[i]
    feat_bf = feat.astype(bf)

    grow = gates_col.reshape(1, 2 * C)
    wcat = jnp.concatenate([ccw[...] * grow[:, :C],
                            rcw[...] * grow[:, C:]], axis=0) * scat[...]
    z = jnp.dot(wcat, feat_bf, preferred_element_type=jnp.float32)
    zb = (z + bcat[...]).astype(bf)
    act = zb * jax.nn.sigmoid(zb)

    logit = conv3x3(act[:C] + feat_bf, cpw) + cpb[...]
    cls = jax.nn.sigmoid(logit)

    r = conv3x3(act[C:], rpw) + rpb[...]
    rows = []
    for f in range(4):
        blk = r[17 * f:17 * (f + 1), :]
        m = jnp.max(blk, axis=0, keepdims=True)
        e = jnp.exp(blk - m)
        s = jnp.sum(e, axis=0, keepdims=True)
        wsum = jnp.sum(e * proj, axis=0, keepdims=True)
        rows.append(wsum / s)
    return cls, jnp.concatenate(rows, axis=0)


def _batched_gates(NB, L, feat_ref, fcwc, fcbc):
    avgs = [jnp.sum(feat_ref[i], axis=1, keepdims=True) * (1.0 / L)
            for i in range(NB)]
    avg2 = avgs[0] if NB == 1 else jnp.concatenate(avgs, axis=1)
    g = jnp.dot(fcwc[...], avg2.astype(jnp.bfloat16),
                preferred_element_type=jnp.float32)
    return jax.nn.sigmoid(g + fcbc[...]).astype(jnp.bfloat16)


def _head_body(H, W, NB, feat_ref, *rest):
    params, (cls_out, reg_out) = rest[:-2], rest[-2:]
    gates = _batched_gates(NB, H * W, feat_ref, params[0], params[1])
    for i in range(NB):
        cls, reg = _scale_compute(H, W, i, gates[:, i:i + 1],
                                  feat_ref, *params)
        cls_out[i] = cls
        reg_out[i] = reg


def _head_body_cat(H, W, off, feat_ref, *rest):
    (c0, r0, c1, r1) = rest[-6:-2]
    cls_out, reg_out = rest[-2:]
    params = rest[:-6]
    gates = _batched_gates(1, H * W, feat_ref, params[0], params[1])
    cls, reg = _scale_compute(H, W, 0, gates, feat_ref, *params)
    l0 = c0.shape[2]
    cls_out[0, :, 0:l0] = c0[0]
    cls_out[0, :, l0:off] = c1[0]
    cls_out[0, :, off:] = cls
    reg_out[0, :, 0:l0] = r0[0]
    reg_out[0, :, l0:off] = r1[0]
    reg_out[0, :, off:] = reg


def _head_scale(feat, ps, H, W, NB=1, carry=None, interpret=False):
    (cfw, cfb, ccw, cbs, cbb, rfw, rfb, rcw, rbs, rbb,
     pcw, pcb, prw, prb) = ps
    B, C, _, _ = feat.shape
    L = H * W
    bf = jnp.bfloat16
    featr = feat.reshape(B, C, L)
    col = lambda v: v.reshape(-1, 1)
    w11 = lambda w: w.reshape(w.shape[0], w.shape[1]).astype(bf)
    w33 = lambda w: w.transpose(3, 0, 2, 1).reshape(
        3, w.shape[0], 3 * w.shape[1]).astype(bf)

    fcw_cat = jnp.concatenate([w11(cfw), w11(rfw)], axis=0)
    fcb_cat = jnp.concatenate([cfb, rfb]).reshape(-1, 1)
    scat = jnp.concatenate([cbs, rbs]).reshape(-1, 1).astype(bf)
    bcat = jnp.concatenate([cbb, rbb]).reshape(-1, 1)

    args = [featr,
            fcw_cat, fcb_cat, w11(ccw), w11(rcw), scat, bcat,
            w33(pcw), col(pcb), w33(prw), col(prb)]

    full = lambda a: pl.BlockSpec(a.shape, lambda b, _n=a.ndim: (0,) * _n)
    in_specs = [pl.BlockSpec((NB, C, L), lambda b: (b, 0, 0))]
    in_specs += [full(a) for a in args[1:]]

    import functools
    if carry is None:
        body = functools.partial(_head_body, H, W, NB)
        out_specs = [pl.BlockSpec((NB, _NC, L), lambda b: (b, 0, 0)),
                     pl.BlockSpec((NB, 4, L), lambda b: (b, 0, 0))]
        out_shape = [jax.ShapeDtypeStruct((B, _NC, L), jnp.float32),
                     jax.ShapeDtypeStruct((B, 4, L), jnp.float32)]
    else:
        c0, r0, c1, r1 = carry
        off = c0.shape[2] + c1.shape[2]
        LT = off + L
        body = functools.partial(_head_body_cat, H, W, off)
        for a in carry:
            args.append(a)
            in_specs.append(
                pl.BlockSpec((1,) + a.shape[1:],
                             lambda b, _n=a.ndim - 1: (b,) + (0,) * _n))
        out_specs = [pl.BlockSpec((1, _NC, LT), lambda b: (b, 0, 0)),
                     pl.BlockSpec((1, 4, LT), lambda b: (b, 0, 0))]
        out_shape = [jax.ShapeDtypeStruct((B, _NC, LT), jnp.float32),
                     jax.ShapeDtypeStruct((B, 4, LT), jnp.float32)]

    cls_s, reg_d = pl.pallas_call(
        body,
        grid=(B // NB,),
        in_specs=in_specs,
        out_specs=out_specs,
        out_shape=out_shape,
        compiler_params=pltpu.CompilerParams(
            dimension_semantics=("arbitrary",)),
        interpret=interpret,
    )(*args)
    return cls_s, reg_d


def _anchors():
    pts, st = [], []
    for (h, w), s in zip(_HWS, _STRIDES):
        sx = jnp.arange(w, dtype=jnp.float32) + _OFFSET
        sy = jnp.arange(h, dtype=jnp.float32) + _OFFSET
        yy, xx = jnp.meshgrid(sy, sx, indexing='ij')
        pts.append(jnp.stack([xx, yy], -1).reshape(-1, 2))
        st.append(jnp.full((h * w, 1), s, dtype=jnp.float32))
    return jnp.concatenate(pts, 0), jnp.concatenate(st, 0)


def kernel(feat0, feat1, feat2,
           p0_cls_fc_w, p0_cls_fc_b, p0_cls_conv_w, p0_cls_bn_s, p0_cls_bn_b,
           p0_reg_fc_w, p0_reg_fc_b, p0_reg_conv_w, p0_reg_bn_s, p0_reg_bn_b,
           p0_cls_pred_w, p0_cls_pred_b, p0_reg_pred_w, p0_reg_pred_b,
           p1_cls_fc_w, p1_cls_fc_b, p1_cls_conv_w, p1_cls_bn_s, p1_cls_bn_b,
           p1_reg_fc_w, p1_reg_fc_b, p1_reg_conv_w, p1_reg_bn_s, p1_reg_bn_b,
           p1_cls_pred_w, p1_cls_pred_b, p1_reg_pred_w, p1_reg_pred_b,
           p2_cls_fc_w, p2_cls_fc_b, p2_cls_conv_w, p2_cls_bn_s, p2_cls_bn_b,
           p2_reg_fc_w, p2_reg_fc_b, p2_reg_conv_w, p2_reg_bn_s, p2_reg_bn_b,
           p2_cls_pred_w, p2_cls_pred_b, p2_reg_pred_w, p2_reg_pred_b):
    feats = (feat0, feat1, feat2)
    params = (
        (p0_cls_fc_w, p0_cls_fc_b, p0_cls_conv_w, p0_cls_bn_s, p0_cls_bn_b,
         p0_reg_fc_w, p0_reg_fc_b, p0_reg_conv_w, p0_reg_bn_s, p0_reg_bn_b,
         p0_cls_pred_w, p0_cls_pred_b, p0_reg_pred_w, p0_reg_pred_b),
        (p1_cls_fc_w, p1_cls_fc_b, p1_cls_conv_w, p1_cls_bn_s, p1_cls_bn_b,
         p1_reg_fc_w, p1_reg_fc_b, p1_reg_conv_w, p1_reg_bn_s, p1_reg_bn_b,
         p1_cls_pred_w, p1_cls_pred_b, p1_reg_pred_w, p1_reg_pred_b),
        (p2_cls_fc_w, p2_cls_fc_b, p2_cls_conv_w, p2_cls_bn_s, p2_cls_bn_b,
         p2_reg_fc_w, p2_reg_fc_b, p2_reg_conv_w, p2_reg_bn_s, p2_reg_bn_b,
         p2_cls_pred_w, p2_cls_pred_b, p2_reg_pred_w, p2_reg_pred_b),
    )
    c0, r0 = _head_scale(feats[0], params[0], *_HWS[0], NB=2)
    c1, r1 = _head_scale(feats[1], params[1], *_HWS[1], NB=2)
    cls_score, reg_dist = _head_scale(feats[2], params[2], *_HWS[2], NB=1,
                                      carry=(c0, r0, c1, r1))
    anchor_points, stride_tensor = _anchors()
    return cls_score, reg_dist, anchor_points, stride_tensor


# speedup vs baseline: 1.0245x; 1.0245x over previous
import jax
import jax.numpy as jnp
from jax.experimental import pallas as pl
from jax.experimental.pallas import tpu as pltpu

_REG_MAX = 16
_NC = 80
_NREG = 4 * (_REG_MAX + 1)
_STRIDES = (32, 16, 8)
_HWS = ((20, 20), (40, 40), (80, 80))
_OFFSET = 0.5


def _scale_compute(H, W, i, gates_col, feat_ref,
                   fcwc, fcbc, ccw, rcw, scat, bcat,
                   cpw, cpb, rpw, rpb):
    C = feat_ref.shape[1]
    L = H * W
    bf = jnp.bfloat16
    cx = jax.lax.broadcasted_iota(jnp.int32, (1, L), 1) % W
    proj = jax.lax.broadcasted_iota(
        jnp.int32, (_REG_MAX + 1, 1), 0).astype(jnp.float32)

    def conv3x3(xb, w_ref):
        zrow = jnp.zeros((C, W), bf)
        x3 = jnp.concatenate([
            jnp.concatenate([zrow, xb[:, :L - W]], axis=1),
            xb,
            jnp.concatenate([xb[:, W:], zrow], axis=1),
        ], axis=0)
        P = [jnp.dot(w_ref[kx], x3, preferred_element_type=jnp.float32)
             for kx in range(3)]
        left = pltpu.roll(P[0], 1, axis=1)
        right = pltpu.roll(P[2], L - 1, axis=1)
        return (P[1] + jnp.where(cx >= 1, left, 0.0)
                + jnp.where(cx <= W - 2, right, 0.0))

    feat = feat_ref[i]
    feat_bf = feat.astype(bf)

    grow = gates_col.reshape(1, 2 * C)
    wcat = jnp.concatenate([ccw[...] * grow[:, :C],
                            rcw[...] * grow[:, C:]], axis=0) * scat[...]
    z = jnp.dot(wcat, feat_bf, preferred_element_type=jnp.float32)
    z = z + bcat[...]
    act = z * jax.nn.sigmoid(z)

    logit = conv3x3(act[:C].astype(bf) + feat_bf, cpw) + cpb[...]
    cls = jax.nn.sigmoid(logit)

    r = conv3x3(act[C:].astype(bf), rpw) + rpb[...]
    rows = []
    for f in range(4):
        blk = r[17 * f:17 * (f + 1), :]
        m = jnp.max(blk, axis=0, keepdims=True)
        e = jnp.exp(blk - m)
        s = jnp.sum(e, axis=0, keepdims=True)
        wsum = jnp.sum(e * proj, axis=0, keepdims=True)
        rows.append(wsum / s)
    return cls, jnp.concatenate(rows, axis=0)


def _batched_gates(NB, L, feat_ref, fcwc, fcbc):
    avgs = [jnp.sum(feat_ref[i], axis=1, keepdims=True) * (1.0 / L)
            for i in range(NB)]
    avg2 = avgs[0] if NB == 1 else jnp.concatenate(avgs, axis=1)
    g = jnp.dot(fcwc[...], avg2.astype(jnp.bfloat16),
                preferred_element_type=jnp.float32)
    return jax.nn.sigmoid(g + fcbc[...]).astype(jnp.bfloat16)


def _head_body(H, W, NB, feat_ref, *rest):
    params, (cls_out, reg_out) = rest[:-2], rest[-2:]
    gates = _batched_gates(NB, H * W, feat_ref, params[0], params[1])
    for i in range(NB):
        cls, reg = _scale_compute(H, W, i, gates[:, i:i + 1],
                                  feat_ref, *params)
        cls_out[i] = cls
        reg_out[i] = reg


def _head_body_cat(H, W, off, feat_ref, *rest):
    (c0, r0, c1, r1) = rest[-6:-2]
    cls_out, reg_out = rest[-2:]
    params = rest[:-6]
    gates = _batched_gates(1, H * W, feat_ref, params[0], params[1])
    cls, reg = _scale_compute(H, W, 0, gates, feat_ref, *params)
    l0 = c0.shape[2]
    cls_out[0, :, 0:l0] = c0[0]
    cls_out[0, :, l0:off] = c1[0]
    cls_out[0, :, off:] = cls
    reg_out[0, :, 0:l0] = r0[0]
    reg_out[0, :, l0:off] = r1[0]
    reg_out[0, :, off:] = reg


def _head_scale(feat, ps, H, W, NB=1, carry=None, interpret=False):
    (cfw, cfb, ccw, cbs, cbb, rfw, rfb, rcw, rbs, rbb,
     pcw, pcb, prw, prb) = ps
    B, C, _, _ = feat.shape
    L = H * W
    bf = jnp.bfloat16
    featr = feat.reshape(B, C, L)
    col = lambda v: v.reshape(-1, 1)
    w11 = lambda w: w.reshape(w.shape[0], w.shape[1]).astype(bf)
    w33 = lambda w: w.transpose(3, 0, 2, 1).reshape(
        3, w.shape[0], 3 * w.shape[1]).astype(bf)

    fcw_cat = jnp.concatenate([w11(cfw), w11(rfw)], axis=0)
    fcb_cat = jnp.concatenate([cfb, rfb]).reshape(-1, 1)
    scat = jnp.concatenate([cbs, rbs]).reshape(-1, 1).astype(bf)
    bcat = jnp.concatenate([cbb, rbb]).reshape(-1, 1)

    args = [featr,
            fcw_cat, fcb_cat, w11(ccw), w11(rcw), scat, bcat,
            w33(pcw), col(pcb), w33(prw), col(prb)]

    full = lambda a: pl.BlockSpec(a.shape, lambda b, _n=a.ndim: (0,) * _n)
    in_specs = [pl.BlockSpec((NB, C, L), lambda b: (b, 0, 0))]
    in_specs += [full(a) for a in args[1:]]

    import functools
    if carry is None:
        body = functools.partial(_head_body, H, W, NB)
        out_specs = [pl.BlockSpec((NB, _NC, L), lambda b: (b, 0, 0)),
                     pl.BlockSpec((NB, 4, L), lambda b: (b, 0, 0))]
        out_shape = [jax.ShapeDtypeStruct((B, _NC, L), jnp.float32),
                     jax.ShapeDtypeStruct((B, 4, L), jnp.float32)]
    else:
        c0, r0, c1, r1 = carry
        off = c0.shape[2] + c1.shape[2]
        LT = off + L
        body = functools.partial(_head_body_cat, H, W, off)
        for a in carry:
            args.append(a)
            in_specs.append(
                pl.BlockSpec((1,) + a.shape[1:],
                             lambda b, _n=a.ndim - 1: (b,) + (0,) * _n))
        out_specs = [pl.BlockSpec((1, _NC, LT), lambda b: (b, 0, 0)),
                     pl.BlockSpec((1, 4, LT), lambda b: (b, 0, 0))]
        out_shape = [jax.ShapeDtypeStruct((B, _NC, LT), jnp.float32),
                     jax.ShapeDtypeStruct((B, 4, LT), jnp.float32)]

    cls_s, reg_d = pl.pallas_call(
        body,
        grid=(B // NB,),
        in_specs=in_specs,
        out_specs=out_specs,
        out_shape=out_shape,
        compiler_params=pltpu.CompilerParams(
            dimension_semantics=("arbitrary",)),
        interpret=interpret,
    )(*args)
    return cls_s, reg_d


def _anchors():
    pts, st = [], []
    for (h, w), s in zip(_HWS, _STRIDES):
        sx = jnp.arange(w, dtype=jnp.float32) + _OFFSET
        sy = jnp.arange(h, dtype=jnp.float32) + _OFFSET
        yy, xx = jnp.meshgrid(sy, sx, indexing='ij')
        pts.append(jnp.stack([xx, yy], -1).reshape(-1, 2))
        st.append(jnp.full((h * w, 1), s, dtype=jnp.float32))
    return jnp.concatenate(pts, 0), jnp.concatenate(st, 0)


def kernel(feat0, feat1, feat2,
           p0_cls_fc_w, p0_cls_fc_b, p0_cls_conv_w, p0_cls_bn_s, p0_cls_bn_b,
           p0_reg_fc_w, p0_reg_fc_b, p0_reg_conv_w, p0_reg_bn_s, p0_reg_bn_b,
           p0_cls_pred_w, p0_cls_pred_b, p0_reg_pred_w, p0_reg_pred_b,
           p1_cls_fc_w, p1_cls_fc_b, p1_cls_conv_w, p1_cls_bn_s, p1_cls_bn_b,
           p1_reg_fc_w, p1_reg_fc_b, p1_reg_conv_w, p1_reg_bn_s, p1_reg_bn_b,
           p1_cls_pred_w, p1_cls_pred_b, p1_reg_pred_w, p1_reg_pred_b,
           p2_cls_fc_w, p2_cls_fc_b, p2_cls_conv_w, p2_cls_bn_s, p2_cls_bn_b,
           p2_reg_fc_w, p2_reg_fc_b, p2_reg_conv_w, p2_reg_bn_s, p2_reg_bn_b,
           p2_cls_pred_w, p2_cls_pred_b, p2_reg_pred_w, p2_reg_pred_b):
    feats = (feat0, feat1, feat2)
    params = (
        (p0_cls_fc_w, p0_cls_fc_b, p0_cls_conv_w, p0_cls_bn_s, p0_cls_bn_b,
         p0_reg_fc_w, p0_reg_fc_b, p0_reg_conv_w, p0_reg_bn_s, p0_reg_bn_b,
         p0_cls_pred_w, p0_cls_pred_b, p0_reg_pred_w, p0_reg_pred_b),
        (p1_cls_fc_w, p1_cls_fc_b, p1_cls_conv_w, p1_cls_bn_s, p1_cls_bn_b,
         p1_reg_fc_w, p1_reg_fc_b, p1_reg_conv_w, p1_reg_bn_s, p1_reg_bn_b,
         p1_cls_pred_w, p1_cls_pred_b, p1_reg_pred_w, p1_reg_pred_b),
        (p2_cls_fc_w, p2_cls_fc_b, p2_cls_conv_w, p2_cls_bn_s, p2_cls_bn_b,
         p2_reg_fc_w, p2_reg_fc_b, p2_reg_conv_w, p2_reg_bn_s, p2_reg_bn_b,
         p2_cls_pred_w, p2_cls_pred_b, p2_reg_pred_w, p2_reg_pred_b),
    )
    c0, r0 = _head_scale(feats[0], params[0], *_HWS[0], NB=4)
    c1, r1 = _head_scale(feats[1], params[1], *_HWS[1], NB=2)
    cls_score, reg_dist = _head_scale(feats[2], params[2], *_HWS[2], NB=1,
                                      carry=(c0, r0, c1, r1))
    anchor_points, stride_tensor = _anchors()
    return cls_score, reg_dist, anchor_points, stride_tensor
